# baseline (speedup 1.0000x reference)
import functools

import jax
import jax.numpy as jnp
from jax import lax
from jax.experimental import pallas as pl
from jax.experimental.pallas import tpu as pltpu
from jax.experimental.pallas import tpu_sc as plsc

VOCAB = 1000000
EMBED_DIM = 64
BATCH = 16384

NUM_CORES = 2
NUM_SUBCORES = 16
LANES = 16
NUM_WORKERS = NUM_CORES * NUM_SUBCORES
BPW = BATCH // NUM_WORKERS
CHUNK = 128
NCHUNK = BPW // CHUNK


def _word2vec_body(tgt_hbm, ctx_hbm, tin_hbm, tout_hbm, out_hbm,
                   tgt_v, ctx_v, trows, crows, out_v, sem_t, sem_c):
    wid = lax.axis_index("s") * NUM_CORES + lax.axis_index("c")
    base = wid * BPW

    for j in range(NCHUNK):
        pltpu.sync_copy(tgt_hbm.at[pl.ds(base + j * CHUNK, CHUNK)], tgt_v.at[j])
        pltpu.sync_copy(ctx_hbm.at[pl.ds(base + j * CHUNK, CHUNK)], ctx_v.at[j])

    copies = []
    for j in range(NCHUNK):
        copies.append(pltpu.async_copy(
            tin_hbm.at[tgt_v.at[j]], trows.at[pl.ds(j * CHUNK, CHUNK)], sem_t))
        copies.append(pltpu.async_copy(
            tout_hbm.at[ctx_v.at[j]], crows.at[pl.ds(j * CHUNK, CHUNK)], sem_c))
    for cp in copies:
        cp.wait()

    lanes = lax.iota(jnp.int32, LANES)

    def block(i, carry):
        r0 = i * LANES
        acc = jnp.zeros((LANES,), jnp.float32)
        for r in range(LANES):
            row = r0 + r
            s = jnp.zeros((LANES,), jnp.float32)
            for k in range(EMBED_DIM // LANES):
                tv = trows[row, pl.ds(k * LANES, LANES)]
                cv = crows[row, pl.ds(k * LANES, LANES)]
                s = s + tv * cv
            tot = jnp.sum(s)
            acc = jnp.where(lanes == r, tot, acc)
        out_v[pl.ds(r0, LANES)] = acc
        return carry

    lax.fori_loop(0, BPW // LANES, block, 0)

    pltpu.sync_copy(out_v, out_hbm.at[pl.ds(base, BPW)])


@jax.jit
def _word2vec(target, context, in_table, out_table):
    mesh = plsc.VectorSubcoreMesh(core_axis_name="c", subcore_axis_name="s")
    return pl.kernel(
        _word2vec_body,
        mesh=mesh,
        compiler_params=pltpu.CompilerParams(
            needs_layout_passes=False, use_tc_tiling_on_sc=False),
        out_type=jax.ShapeDtypeStruct((BATCH,), jnp.float32),
        scratch_types=[
            pltpu.VMEM((NCHUNK, CHUNK), jnp.int32),
            pltpu.VMEM((NCHUNK, CHUNK), jnp.int32),
            pltpu.VMEM((BPW, EMBED_DIM), jnp.float32),
            pltpu.VMEM((BPW, EMBED_DIM), jnp.float32),
            pltpu.VMEM((BPW,), jnp.float32),
            pltpu.SemaphoreType.DMA,
            pltpu.SemaphoreType.DMA,
        ],
    )(target, context, in_table, out_table)


def kernel(target, context, in_table, out_table):
    return _word2vec(target.astype(jnp.int32), context.astype(jnp.int32),
                     in_table, out_table)
